# baseline (speedup 1.0000x reference)
import functools

import jax
import jax.numpy as jnp
from jax import lax
from jax.experimental import pallas as pl
from jax.experimental.pallas import tpu as pltpu
from jax.experimental.pallas import tpu_sc as plsc

NUM_EMBEDDINGS = 1000000
EMBEDDING_DIM = 32

NC = 2
NS = 16
NW = NC * NS

NTC = NUM_EMBEDDINGS // 128
LIN_WORDS = NUM_EMBEDDINGS * EMBEDDING_DIM
LIN_ROWS = NUM_EMBEDDINGS // 4

BATCH = 4096
SEQ = 50
UNITS = SEQ * (BATCH // 128)
UNITS_PER_W = UNITS // NW
TCC = 32768
TCQ_SHIFT = 13
IDX_PER_W = UNITS_PER_W * 128


def _iota16():
    return lax.iota(jnp.int32, 16)


def _detile_kernel(wt_hbm, ltail_hbm, lin_hbm, vin, vout, gsems, ssems):
    wid = lax.axis_index("s") * NC + lax.axis_index("c")

    def col_off(rt):
        return pl.multiple_of(rt * 128, 128)

    def lin_off(rt):
        return pl.multiple_of(rt * 4096, 4096)

    def start_reads(rt, b):
        for jt in range(4):
            pltpu.async_copy(
                wt_hbm.at[pl.ds(8 * jt, 8), pl.ds(col_off(rt), 128)],
                vin.at[b, jt],
                gsems.at[b],
            )

    def drain_reads(rt, b):
        for jt in range(4):
            pltpu.make_async_copy(
                wt_hbm.at[pl.ds(8 * jt, 8), pl.ds(col_off(rt), 128)],
                vin.at[b, jt],
                gsems.at[b],
            ).wait()

    def drain_store(rt, b):
        pltpu.make_async_copy(
            vout.at[b], lin_hbm.at[pl.ds(lin_off(rt), 4096)], ssems.at[b]
        ).wait()

    scatv = _iota16() * 32

    def transpose(b):
        bv = jnp.full((16,), 0, jnp.int32) + b
        for jt in range(4):
            for js in range(8):
                j = 8 * jt + js
                for k in range(8):
                    v = vin[b, jt, js, pl.ds(16 * k, 16)]
                    plsc.store_scatter(vout, [bv, scatv + (512 * k + j)], v)

    n_units = 244 + jnp.where(wid < 4, 1, 0)

    start_reads(wid, 0)

    def step(k, _):
        rt = wid + 32 * k
        b = k % 2

        @pl.when(k + 1 < n_units)
        def _():
            start_reads(wid + 32 * (k + 1), 1 - b)

        drain_reads(rt, b)

        @pl.when(k >= 2)
        def _():
            drain_store(wid + 32 * (k - 2), b)

        transpose(b)
        pltpu.async_copy(
            vout.at[b], lin_hbm.at[pl.ds(lin_off(rt), 4096)], ssems.at[b]
        )
        return ()

    lax.fori_loop(0, n_units, step, ())

    drain_store(wid + 32 * (n_units - 2), n_units % 2)
    drain_store(wid + 32 * (n_units - 1), (n_units - 1) % 2)

    @pl.when(wid == 31)
    def _():
        pltpu.sync_copy(ltail_hbm, vout.at[0, pl.ds(0, 2048)])
        pltpu.sync_copy(
            vout.at[0, pl.ds(0, 2048)], lin_hbm.at[pl.ds(NTC * 4096, 2048)]
        )


def _gather_kernel(lin_hbm, idx_hbm, out_hbm, idx_v, mv, gv, vout, gsems, osems):
    wid = lax.axis_index("s") * NC + lax.axis_index("c")
    u0 = wid * UNITS_PER_W
    pltpu.sync_copy(idx_hbm.at[pl.ds(u0 * 128, IDX_PER_W)], idx_v)

    CH_ROWS = 1280

    def compute_mv(c, cb):
        off0 = pl.multiple_of(c * CH_ROWS, 128)
        cbv = jnp.full((16,), 0, jnp.int32) + cb
        for blk in range(CH_ROWS // 16):
            r = idx_v[pl.ds(off0 + blk * 16, 16)]
            c2 = r & (TCC - 1)
            rho = (r - c2) + ((c2 & (TCC // 4 - 1)) << 2) + (c2 >> TCQ_SHIFT)
            plsc.store_scatter(
                mv, [cbv, _iota16() + (blk * 16)], rho
            )

    def start_gather(cb):
        pltpu.async_copy(lin_hbm.at[mv.at[cb]], gv.at[cb], gsems.at[cb])

    def drain_gather(cb):
        pltpu.make_async_copy(lin_hbm.at[mv.at[cb]], gv.at[cb], gsems.at[cb]).wait()

    def out_copies(i, b2, make_only):
        u = u0 + i
        s, bt = u >> 5, u & 31
        off = pl.multiple_of((((s * 4)) * 32 + bt) * 1024, 1024)
        for jt in range(4):
            src = vout.at[b2, pl.ds(jt * 1024, 1024)]
            dst = out_hbm.at[pl.ds(off + jt * 32768, 1024)]
            if make_only:
                pltpu.make_async_copy(src, dst, osems.at[b2]).wait()
            else:
                pltpu.async_copy(src, dst, osems.at[b2])

    ibase = [_iota16() * 128, (_iota16() + 16) * 128]

    def transpose_select(c, ul, b2):
        cb = c % 2
        row0 = ul * 128
        b2v = jnp.full((16,), 0, jnp.int32) + b2
        for bs in range(128):
            for h in range(2):
                v = gv[cb, row0 + bs, pl.ds(16 * h, 16)]
                plsc.store_scatter(vout, [b2v, ibase[h] + bs], v)

    compute_mv(0, 0)
    start_gather(0)

    def chunk_step(c, _):
        cb = c % 2

        @pl.when(c + 1 < 5)
        def _():
            compute_mv(c + 1, 1 - cb)
            start_gather(1 - cb)

        drain_gather(cb)

        def unit_pair(kk, _):
            for b2 in range(2):
                ul = 2 * kk + b2
                i = c * 10 + ul

                @pl.when(i >= 2)
                def _():
                    out_copies(i - 2, b2, make_only=True)

                transpose_select(c, ul, b2)
                out_copies(i, b2, make_only=False)
            return ()

        lax.fori_loop(0, 5, unit_pair, ())
        return ()

    lax.fori_loop(0, 5, chunk_step, ())
    out_copies(UNITS_PER_W - 2, 0, make_only=True)
    out_copies(UNITS_PER_W - 1, 1, make_only=True)


def _mesh():
    return plsc.VectorSubcoreMesh(core_axis_name="c", subcore_axis_name="s")


TCG = (NUM_EMBEDDINGS + TCC - 1) // TCC
LIN_PAD_ROWS = TCG * (TCC // 4)


def _tc_detile_kernel(wt_ref, lin_ref):
    x = wt_ref[...]
    parts = [x[:, g * (TCC // 4):(g + 1) * (TCC // 4)].T for g in range(4)]
    lin_ref[...] = jnp.concatenate(parts, axis=1)


def _tc_detile(wt):
    return pl.pallas_call(
        _tc_detile_kernel,
        grid=(TCG,),
        in_specs=[pl.BlockSpec((32, TCC), lambda i: (0, i))],
        out_specs=pl.BlockSpec((TCC // 4, 128), lambda i: (i, 0)),
        out_shape=jax.ShapeDtypeStruct((LIN_PAD_ROWS, 128), jnp.float32),
    )(wt)


@jax.jit
def _emb_lookup(x, weight):
    wt = weight.T
    ltail = weight[NTC * 128:].reshape(-1)
    idx_flat = x.T.reshape(-1)

    lin2d = _tc_detile(wt).reshape(LIN_PAD_ROWS * 4, EMBEDDING_DIM)

    gather = functools.partial(
        pl.kernel,
        mesh=_mesh(),
        out_type=jax.ShapeDtypeStruct((SEQ * 4 * (BATCH // 128) * 1024,), jnp.float32),
        scratch_types=[
            pltpu.VMEM((IDX_PER_W,), jnp.int32),
            pltpu.VMEM((2, 1280), jnp.int32),
            pltpu.VMEM((2, 1280, EMBEDDING_DIM), jnp.float32),
            pltpu.VMEM((2, 4096), jnp.float32),
            pltpu.SemaphoreType.DMA((2,)),
            pltpu.SemaphoreType.DMA((2,)),
        ],
        compiler_params=pltpu.CompilerParams(
            use_tc_tiling_on_sc=False, needs_layout_passes=False
        ),
    )(_gather_kernel)
    out5 = gather(lin2d, idx_flat).reshape(SEQ, 4, BATCH // 128, 8, 128)
    out = out5.transpose(2, 4, 0, 1, 3).reshape(BATCH, SEQ, EMBEDDING_DIM)
    return out


def kernel(x, weight):
    return _emb_lookup(x, weight)


# speedup vs baseline: 1.0014x; 1.0014x over previous
import functools

import jax
import jax.numpy as jnp
from jax import lax
from jax.experimental import pallas as pl
from jax.experimental.pallas import tpu as pltpu
from jax.experimental.pallas import tpu_sc as plsc

NUM_EMBEDDINGS = 1000000
EMBEDDING_DIM = 32

NC = 2
NS = 16
NW = NC * NS

NTC = NUM_EMBEDDINGS // 128
LIN_WORDS = NUM_EMBEDDINGS * EMBEDDING_DIM
LIN_ROWS = NUM_EMBEDDINGS // 4

BATCH = 4096
SEQ = 50
UNITS = SEQ * (BATCH // 128)
UNITS_PER_W = UNITS // NW
TCC = 32768
TCQ_SHIFT = 13
IDX_PER_W = UNITS_PER_W * 128


def _iota16():
    return lax.iota(jnp.int32, 16)


def _detile_kernel(wt_hbm, ltail_hbm, lin_hbm, vin, vout, gsems, ssems):
    wid = lax.axis_index("s") * NC + lax.axis_index("c")

    def col_off(rt):
        return pl.multiple_of(rt * 128, 128)

    def lin_off(rt):
        return pl.multiple_of(rt * 4096, 4096)

    def start_reads(rt, b):
        for jt in range(4):
            pltpu.async_copy(
                wt_hbm.at[pl.ds(8 * jt, 8), pl.ds(col_off(rt), 128)],
                vin.at[b, jt],
                gsems.at[b],
            )

    def drain_reads(rt, b):
        for jt in range(4):
            pltpu.make_async_copy(
                wt_hbm.at[pl.ds(8 * jt, 8), pl.ds(col_off(rt), 128)],
                vin.at[b, jt],
                gsems.at[b],
            ).wait()

    def drain_store(rt, b):
        pltpu.make_async_copy(
            vout.at[b], lin_hbm.at[pl.ds(lin_off(rt), 4096)], ssems.at[b]
        ).wait()

    scatv = _iota16() * 32

    def transpose(b):
        bv = jnp.full((16,), 0, jnp.int32) + b
        for jt in range(4):
            for js in range(8):
                j = 8 * jt + js
                for k in range(8):
                    v = vin[b, jt, js, pl.ds(16 * k, 16)]
                    plsc.store_scatter(vout, [bv, scatv + (512 * k + j)], v)

    n_units = 244 + jnp.where(wid < 4, 1, 0)

    start_reads(wid, 0)

    def step(k, _):
        rt = wid + 32 * k
        b = k % 2

        @pl.when(k + 1 < n_units)
        def _():
            start_reads(wid + 32 * (k + 1), 1 - b)

        drain_reads(rt, b)

        @pl.when(k >= 2)
        def _():
            drain_store(wid + 32 * (k - 2), b)

        transpose(b)
        pltpu.async_copy(
            vout.at[b], lin_hbm.at[pl.ds(lin_off(rt), 4096)], ssems.at[b]
        )
        return ()

    lax.fori_loop(0, n_units, step, ())

    drain_store(wid + 32 * (n_units - 2), n_units % 2)
    drain_store(wid + 32 * (n_units - 1), (n_units - 1) % 2)

    @pl.when(wid == 31)
    def _():
        pltpu.sync_copy(ltail_hbm, vout.at[0, pl.ds(0, 2048)])
        pltpu.sync_copy(
            vout.at[0, pl.ds(0, 2048)], lin_hbm.at[pl.ds(NTC * 4096, 2048)]
        )


def _gather_kernel(lin_hbm, idx_hbm, out_hbm, idx_v, mv, gv, vout, gsems, osems):
    wid = lax.axis_index("s") * NC + lax.axis_index("c")
    u0 = wid * UNITS_PER_W
    pltpu.sync_copy(idx_hbm.at[pl.ds(u0 * 128, IDX_PER_W)], idx_v)

    CH_ROWS = 1280

    def compute_mv(c, cb):
        off0 = pl.multiple_of(c * CH_ROWS, 128)
        cbv = jnp.full((16,), 0, jnp.int32) + cb
        for blk in range(CH_ROWS // 16):
            r = idx_v[pl.ds(off0 + blk * 16, 16)]
            c2 = r & (TCC - 1)
            rho = (r - c2) + ((c2 & (TCC // 4 - 1)) << 2) + (c2 >> TCQ_SHIFT)
            plsc.store_scatter(
                mv, [cbv, _iota16() + (blk * 16)], rho
            )

    def start_gather(cb):
        pltpu.async_copy(lin_hbm.at[mv.at[cb]], gv.at[cb], gsems.at[cb])

    def drain_gather(cb):
        pltpu.make_async_copy(lin_hbm.at[mv.at[cb]], gv.at[cb], gsems.at[cb]).wait()

    def out_copies(i, b2, make_only):
        u = u0 + i
        s, bt = u >> 5, u & 31
        off = pl.multiple_of((((s * 4)) * 32 + bt) * 1024, 1024)
        for jt in range(4):
            src = vout.at[pl.ds(b2 * 4096 + jt * 1024, 1024)]
            dst = out_hbm.at[pl.ds(off + jt * 32768, 1024)]
            if make_only:
                pltpu.make_async_copy(src, dst, osems.at[b2]).wait()
            else:
                pltpu.async_copy(src, dst, osems.at[b2])

    ibase = [_iota16() * 128, (_iota16() + 16) * 128]

    def transpose_select(c, ul, b2):
        cb = c % 2
        row0 = ul * 128
        for bs in range(128):
            for h in range(2):
                v = gv[cb, row0 + bs, pl.ds(16 * h, 16)]
                plsc.store_scatter(vout, [ibase[h] + (b2 * 4096 + bs)], v)

    compute_mv(0, 0)
    start_gather(0)

    def chunk_step(c, _):
        cb = c % 2

        @pl.when(c + 1 < 5)
        def _():
            compute_mv(c + 1, 1 - cb)
            start_gather(1 - cb)

        drain_gather(cb)

        def unit_pair(kk, _):
            for b2 in range(2):
                ul = 2 * kk + b2
                i = c * 10 + ul

                @pl.when(i >= 2)
                def _():
                    out_copies(i - 2, b2, make_only=True)

                transpose_select(c, ul, b2)
                out_copies(i, b2, make_only=False)
            return ()

        lax.fori_loop(0, 5, unit_pair, ())
        return ()

    lax.fori_loop(0, 5, chunk_step, ())
    out_copies(UNITS_PER_W - 2, 0, make_only=True)
    out_copies(UNITS_PER_W - 1, 1, make_only=True)


def _mesh():
    return plsc.VectorSubcoreMesh(core_axis_name="c", subcore_axis_name="s")


TCG = (NUM_EMBEDDINGS + TCC - 1) // TCC
LIN_PAD_ROWS = TCG * (TCC // 4)


def _tc_detile_kernel(wt_ref, lin_ref):
    x = wt_ref[...]
    parts = [x[:, g * (TCC // 4):(g + 1) * (TCC // 4)].T for g in range(4)]
    lin_ref[...] = jnp.concatenate(parts, axis=1)


def _tc_detile(wt):
    return pl.pallas_call(
        _tc_detile_kernel,
        grid=(TCG,),
        in_specs=[pl.BlockSpec((32, TCC), lambda i: (0, i))],
        out_specs=pl.BlockSpec((TCC // 4, 128), lambda i: (i, 0)),
        out_shape=jax.ShapeDtypeStruct((LIN_PAD_ROWS, 128), jnp.float32),
    )(wt)


@jax.jit
def _emb_lookup(x, weight):
    wt = weight.T
    ltail = weight[NTC * 128:].reshape(-1)
    idx_flat = x.T.reshape(-1)

    lin2d = _tc_detile(wt).reshape(LIN_PAD_ROWS * 4, EMBEDDING_DIM)

    gather = functools.partial(
        pl.kernel,
        mesh=_mesh(),
        out_type=jax.ShapeDtypeStruct((SEQ * 4 * (BATCH // 128) * 1024,), jnp.float32),
        scratch_types=[
            pltpu.VMEM((IDX_PER_W,), jnp.int32),
            pltpu.VMEM((2, 1280), jnp.int32),
            pltpu.VMEM((2, 1280, EMBEDDING_DIM), jnp.float32),
            pltpu.VMEM((8192,), jnp.float32),
            pltpu.SemaphoreType.DMA((2,)),
            pltpu.SemaphoreType.DMA((2,)),
        ],
        compiler_params=pltpu.CompilerParams(
            use_tc_tiling_on_sc=False, needs_layout_passes=False
        ),
    )(_gather_kernel)
    out5 = gather(lin2d, idx_flat).reshape(SEQ, 4, BATCH // 128, 8, 128)
    out = out5.transpose(2, 4, 0, 1, 3).reshape(BATCH, SEQ, EMBEDDING_DIM)
    return out


def kernel(x, weight):
    return _emb_lookup(x, weight)


# speedup vs baseline: 1.0109x; 1.0095x over previous
import functools

import jax
import jax.numpy as jnp
from jax import lax
from jax.experimental import pallas as pl
from jax.experimental.pallas import tpu as pltpu
from jax.experimental.pallas import tpu_sc as plsc

NUM_EMBEDDINGS = 1000000
EMBEDDING_DIM = 32

NC = 2
NS = 16
NW = NC * NS


BATCH = 4096
SEQ = 50
UNITS = SEQ * (BATCH // 128)
UNITS_PER_W = UNITS // NW
TCC = 32768
TCQ_SHIFT = 13
IDX_PER_W = UNITS_PER_W * 128


def _iota16():
    return lax.iota(jnp.int32, 16)


def _gather_kernel(lin_hbm, idx_hbm, out_hbm, idx_v, mv, gv, vout, gsems, osems):
    wid = lax.axis_index("s") * NC + lax.axis_index("c")
    u0 = wid * UNITS_PER_W
    pltpu.sync_copy(idx_hbm.at[pl.ds(u0 * 128, IDX_PER_W)], idx_v)

    def unit_su(i):
        u = u0 + i
        return u >> 5, u & 31

    def compute_mv(i, b):
        off0 = pl.multiple_of(i * 128, 128)
        for blk in range(8):
            r = idx_v[pl.ds(off0 + blk * 16, 16)]
            c2 = r & (TCC - 1)
            rho = (r - c2) + ((c2 & (TCC // 4 - 1)) << 2) + (c2 >> TCQ_SHIFT)
            mv[b, pl.ds(blk * 16, 16)] = rho

    def start_gather(b):
        pltpu.async_copy(lin_hbm.at[mv.at[b]], gv.at[b], gsems.at[b])

    def drain_gather(b):
        pltpu.make_async_copy(lin_hbm.at[mv.at[b]], gv.at[b], gsems.at[b]).wait()

    def out_copies(i, b, make_only):
        s, bt = unit_su(i)
        for jt in range(4):
            src = vout.at[b, pl.ds(jt * 1024, 1024)]
            off = pl.multiple_of((((s * 4) + jt) * 32 + bt) * 1024, 1024)
            dst = out_hbm.at[pl.ds(off, 1024)]
            if make_only:
                pltpu.make_async_copy(src, dst, osems.at[b]).wait()
            else:
                pltpu.async_copy(src, dst, osems.at[b])

    ibase = [_iota16() * 128, (_iota16() + 16) * 128]

    def transpose_select(b):
        bv = jnp.full((16,), b, jnp.int32)
        for bs in range(128):
            for h in range(2):
                v = gv[b, bs, pl.ds(16 * h, 16)]
                plsc.store_scatter(vout, [bv, ibase[h] + bs], v)

    compute_mv(0, 0)
    start_gather(0)

    def step(k, _):
        for b in range(2):
            i = 2 * k + b

            @pl.when(i + 1 < UNITS_PER_W)
            def _():
                compute_mv(i + 1, 1 - b)
                start_gather(1 - b)

            drain_gather(b)

            @pl.when(i >= 2)
            def _():
                out_copies(i - 2, b, make_only=True)

            transpose_select(b)
            out_copies(i, b, make_only=False)
        return ()

    lax.fori_loop(0, UNITS_PER_W // 2, step, ())
    out_copies(UNITS_PER_W - 2, 0, make_only=True)
    out_copies(UNITS_PER_W - 1, 1, make_only=True)


def _mesh():
    return plsc.VectorSubcoreMesh(core_axis_name="c", subcore_axis_name="s")


TCG = (NUM_EMBEDDINGS + TCC - 1) // TCC
LIN_PAD_ROWS = TCG * (TCC // 4)


def _tc_detile_kernel(wt_ref, lin_ref):
    x = wt_ref[...]
    parts = [x[:, g * (TCC // 4):(g + 1) * (TCC // 4)].T for g in range(4)]
    lin_ref[...] = jnp.concatenate(parts, axis=1)


def _tc_detile(wt):
    return pl.pallas_call(
        _tc_detile_kernel,
        grid=(TCG,),
        in_specs=[pl.BlockSpec((32, TCC), lambda i: (0, i))],
        out_specs=pl.BlockSpec((TCC // 4, 128), lambda i: (i, 0)),
        out_shape=jax.ShapeDtypeStruct((LIN_PAD_ROWS, 128), jnp.float32),
    )(wt)


@jax.jit
def _emb_lookup(x, weight):
    wt = weight.T
    idx_flat = x.T.reshape(-1)

    lin2d = _tc_detile(wt).reshape(LIN_PAD_ROWS * 4, EMBEDDING_DIM)

    gather = functools.partial(
        pl.kernel,
        mesh=_mesh(),
        out_type=jax.ShapeDtypeStruct((SEQ * 4 * (BATCH // 128) * 1024,), jnp.float32),
        scratch_types=[
            pltpu.VMEM((IDX_PER_W,), jnp.int32),
            pltpu.VMEM((2, 128), jnp.int32),
            pltpu.VMEM((2, 128, EMBEDDING_DIM), jnp.float32),
            pltpu.VMEM((2, 4096), jnp.float32),
            pltpu.SemaphoreType.DMA((2,)),
            pltpu.SemaphoreType.DMA((2,)),
        ],
        compiler_params=pltpu.CompilerParams(
            use_tc_tiling_on_sc=False, needs_layout_passes=False
        ),
    )(_gather_kernel)
    out5 = gather(lin2d, idx_flat).reshape(SEQ, 4, BATCH // 128, 8, 128)
    out = out5.transpose(2, 4, 0, 1, 3).reshape(BATCH, SEQ, EMBEDDING_DIM)
    return out


def kernel(x, weight):
    return _emb_lookup(x, weight)
